# baseline (speedup 1.0000x reference)
import functools
import math

import jax
import jax.numpy as jnp
import numpy as np
from jax import lax
from jax.experimental import pallas as pl
from jax.experimental.pallas import tpu as pltpu
from jax.experimental.pallas import tpu_sc as plsc

_N_BINS = 18
_EPS = 1e-9
_T = 1024
_LANES = 16
_CHUNKS = _T // _LANES
_N_ROWS = 9
_ROW_WORDS = _N_ROWS * _N_BINS
_LANE_STRIDE = 256
_TASK_WORDS = _LANES * _LANE_STRIDE
_N_TASKS = 128
_UNROLL = 4


def _sc_hist_kernel(pha_hbm, amp_hbm, cut_hbm, out_hbm,
                    pha_v, amp_v, hist_v, cut_v):
    nc = 2
    wid = lax.axis_index("s") * nc + lax.axis_index("c")
    tasks_per_w = _N_TASKS // 32
    base_task = wid * tasks_per_w
    bc = base_task // 16

    pltpu.sync_copy(cut_hbm, cut_v)
    pltpu.sync_copy(amp_hbm.at[pl.ds(bc * 16, 16)], amp_v)
    pltpu.sync_copy(pha_hbm.at[pl.ds(base_task, tasks_per_w)], pha_v)

    lane_off = lax.iota(jnp.int32, _LANES) * _LANE_STRIDE
    scale = jnp.float32(_N_BINS / (2.0 * math.pi))
    pi32 = jnp.float32(math.pi)
    ones = jnp.full((_LANES,), 1.0, dtype=jnp.float32)
    zeros = jnp.zeros((_LANES,), dtype=jnp.float32)

    def _pair_body(tp, c):
        for s in range(2):
            tt = tp * 2 + s
            t_off = tt * _TASK_WORDS

            def _zero_body(i, c2):
                off = t_off + i * _LANE_STRIDE
                for k in range(11):
                    hist_v[pl.ds(off + k * _LANES, _LANES)] = zeros
                return c2
            lax.fori_loop(0, _LANES, _zero_body, 0)

            def _one_chunk(i):
                ph = pha_v[tt, pl.ds(i * _LANES, _LANES)]
                raw = (ph + pi32) * scale
                raw = jnp.minimum(jnp.maximum(raw, 0.0), 17.0)
                idx0 = raw.astype(jnp.int32)
                c_lo = plsc.load_gather(cut_v, [idx0])
                c_hi = plsc.load_gather(cut_v, [idx0 + 1])
                dec = jnp.logical_and(ph <= c_lo, idx0 > 0).astype(jnp.int32)
                inc = jnp.logical_and(ph > c_hi, idx0 < 17).astype(jnp.int32)
                base = (lane_off + t_off) + (idx0 - dec + inc)
                for fa in range(8):
                    av = amp_v[fa * 2 + s, pl.ds(i * _LANES, _LANES)]
                    plsc.addupdate_scatter(hist_v, [base + fa * _N_BINS], av)
                plsc.addupdate_scatter(hist_v, [base + 8 * _N_BINS], ones)

            def _chunk_body(j, c2):
                for k in range(_UNROLL):
                    _one_chunk(j * _UNROLL + k)
                return c2
            lax.fori_loop(0, _CHUNKS // _UNROLL, _chunk_body, 0)
        return c

    lax.fori_loop(0, tasks_per_w // 2, _pair_body, 0)

    pltpu.sync_copy(
        hist_v, out_hbm.at[pl.ds(base_task * _TASK_WORDS,
                                 tasks_per_w * _TASK_WORDS)])


def _sc_hist(pha_r, amp_r, cut_pad):
    mesh = plsc.VectorSubcoreMesh(core_axis_name="c", subcore_axis_name="s")
    f = functools.partial(
        pl.kernel,
        mesh=mesh,
        out_type=jax.ShapeDtypeStruct((_N_TASKS * _TASK_WORDS,), jnp.float32),
        scratch_types=[
            pltpu.VMEM((_N_TASKS // 32, _T), jnp.float32),
            pltpu.VMEM((16, _T), jnp.float32),
            pltpu.VMEM(((_N_TASKS // 32) * _TASK_WORDS,), jnp.float32),
            pltpu.VMEM((24,), jnp.float32),
        ],
        compiler_params=pltpu.CompilerParams(needs_layout_passes=False),
    )(_sc_hist_kernel)
    return f(pha_r, amp_r, cut_pad)


def _tc_finish_kernel(h_ref, o_ref):
    h = h_ref[...]
    acc = h[:, 0:_ROW_WORDS]
    for l in range(1, _LANES):
        off = l * _LANE_STRIDE
        acc = acc + h[:, off:off + _ROW_WORDS]
    counts = acc[:, 8 * _N_BINS:9 * _N_BINS]
    log_n = jnp.float32(np.log(float(_N_BINS)))
    cols = []
    for fa in range(8):
        s_fa = acc[:, fa * _N_BINS:(fa + 1) * _N_BINS]
        mean = s_fa / (counts + _EPS)
        tot = jnp.sum(mean, axis=-1, keepdims=True)
        probs = mean / (tot + _EPS)
        kl = jnp.sum(probs * jnp.log(probs + _EPS), axis=-1, keepdims=True)
        cols.append((log_n + kl) / log_n)
    mi = jnp.concatenate(cols, axis=-1)
    r_id = lax.broadcasted_iota(jnp.int32, (64, 128), 0)
    c_id = lax.broadcasted_iota(jnp.int32, (64, 128), 1)
    sel = jnp.where((c_id >> 1) == r_id, 0.5, 0.0).astype(jnp.float32)
    mi = jnp.dot(sel, mi, preferred_element_type=jnp.float32)
    o_ref[...] = jnp.nan_to_num(mi, nan=0.0)


def _tc_finish(h2):
    return pl.pallas_call(
        _tc_finish_kernel,
        out_shape=jax.ShapeDtypeStruct((64, 8), jnp.float32),
    )(h2)


def kernel(pha, amp):
    pha = pha.astype(jnp.float32)
    amp = amp.astype(jnp.float32)
    pha_r = pha.reshape(_N_TASKS, _T)
    amp_r = amp.reshape(_N_TASKS, _T)
    cutoffs = jnp.linspace(-np.pi, np.pi, _N_BINS + 1).astype(jnp.float32)
    cut_pad = jnp.concatenate([cutoffs, jnp.zeros((5,), jnp.float32)])
    hist = _sc_hist(pha_r, amp_r, cut_pad)
    mi = _tc_finish(hist.reshape(_N_TASKS, _TASK_WORDS))
    return mi.reshape(2, 4, 8, 8)


# speedup vs baseline: 1.0739x; 1.0739x over previous
import functools
import math

import jax
import jax.numpy as jnp
import numpy as np
from jax import lax
from jax.experimental import pallas as pl
from jax.experimental.pallas import tpu as pltpu
from jax.experimental.pallas import tpu_sc as plsc

_N_BINS = 18
_EPS = 1e-9
_T = 1024
_LANES = 16
_CHUNKS = _T // _LANES
_N_ROWS = 9
_ROW_WORDS = _N_ROWS * _N_BINS
_LANE_STRIDE = 256
_TASK_WORDS = _LANES * _LANE_STRIDE
_N_TASKS = 128
_N_SBC = 16
_TASKS_PER_W = _N_TASKS // 32
_UNROLL = 4


def _sc_hist_kernel(pha_hbm, amp_hbm, cut_hbm, out_hbm,
                    pha_v, amp_v, hist_v, cut_v, sem):
    nc = 2
    wid = lax.axis_index("s") * nc + lax.axis_index("c")
    base_task = wid * _TASKS_PER_W
    sbc = base_task // 8

    pltpu.sync_copy(cut_hbm, cut_v)
    pltpu.sync_copy(amp_hbm.at[sbc], amp_v)
    pltpu.sync_copy(pha_hbm.at[pl.ds(base_task, _TASKS_PER_W)], pha_v)

    lane_off = lax.iota(jnp.int32, _LANES) * _LANE_STRIDE
    scale = jnp.float32(_N_BINS / (2.0 * math.pi))
    pi32 = jnp.float32(math.pi)
    ones = jnp.full((_LANES,), 1.0, dtype=jnp.float32)
    zeros = jnp.zeros((_LANES,), dtype=jnp.float32)

    copies = []
    for tt in range(_TASKS_PER_W):
        t_off = tt * _TASK_WORDS

        def _zero_body(i, c2, t_off=t_off):
            off = t_off + i * _LANE_STRIDE
            for k in range(11):
                hist_v[pl.ds(off + k * _LANES, _LANES)] = zeros
            return c2
        lax.fori_loop(0, _LANES, _zero_body, 0)

        base_v = lane_off + t_off

        def _one_chunk(i, tt=tt, base_v=base_v):
            ph = pha_v[tt, pl.ds(i * _LANES, _LANES)]
            raw = (ph + pi32) * scale
            raw = jnp.minimum(jnp.maximum(raw, 0.0), 17.0)
            idx0 = raw.astype(jnp.int32)
            c_lo = plsc.load_gather(cut_v, [idx0])
            c_hi = plsc.load_gather(cut_v, [idx0 + 1])
            dec = jnp.logical_and(ph <= c_lo, idx0 > 0).astype(jnp.int32)
            inc = jnp.logical_and(ph > c_hi, idx0 < 17).astype(jnp.int32)
            base = base_v + (idx0 - dec + inc)
            for fa in range(8):
                av = amp_v[fa, pl.ds(i * _LANES, _LANES)]
                plsc.addupdate_scatter(hist_v, [base + fa * _N_BINS], av)
            plsc.addupdate_scatter(hist_v, [base + 8 * _N_BINS], ones)

        def _chunk_body(j, c2):
            for k in range(_UNROLL):
                _one_chunk(j * _UNROLL + k)
            return c2
        lax.fori_loop(0, _CHUNKS // _UNROLL, _chunk_body, 0)

        copies.append(pltpu.async_copy(
            hist_v.at[pl.ds(t_off, _TASK_WORDS)],
            out_hbm.at[base_task + tt], sem))
    for cp in copies:
        cp.wait()


def _sc_hist(pha_t, amp_t, cut_pad):
    mesh = plsc.VectorSubcoreMesh(core_axis_name="c", subcore_axis_name="s")
    f = functools.partial(
        pl.kernel,
        mesh=mesh,
        out_type=jax.ShapeDtypeStruct((_N_TASKS, _TASK_WORDS), jnp.float32),
        scratch_types=[
            pltpu.VMEM((_TASKS_PER_W, _T), jnp.float32),
            pltpu.VMEM((8, _T), jnp.float32),
            pltpu.VMEM((_TASKS_PER_W * _TASK_WORDS,), jnp.float32),
            pltpu.VMEM((24,), jnp.float32),
            pltpu.SemaphoreType.DMA,
        ],
        compiler_params=pltpu.CompilerParams(needs_layout_passes=False),
    )(_sc_hist_kernel)
    return f(pha_t, amp_t, cut_pad)


def _tc_finish_kernel(h_ref, o_ref):
    h = h_ref[...]
    acc = h[:, 0:_ROW_WORDS]
    for l in range(1, _LANES):
        off = l * _LANE_STRIDE
        acc = acc + h[:, off:off + _ROW_WORDS]
    counts = acc[:, 8 * _N_BINS:9 * _N_BINS]
    log_n = jnp.float32(np.log(float(_N_BINS)))
    cols = []
    for fa in range(8):
        s_fa = acc[:, fa * _N_BINS:(fa + 1) * _N_BINS]
        mean = s_fa / (counts + _EPS)
        tot = jnp.sum(mean, axis=-1, keepdims=True)
        probs = mean / (tot + _EPS)
        kl = jnp.sum(probs * jnp.log(probs + _EPS), axis=-1, keepdims=True)
        cols.append((log_n + kl) / log_n)
    mi = jnp.concatenate(cols, axis=-1)
    mi = 0.5 * (mi[0:64, :] + mi[64:128, :])
    o_ref[...] = jnp.nan_to_num(mi, nan=0.0)


def _tc_finish(h2):
    return pl.pallas_call(
        _tc_finish_kernel,
        out_shape=jax.ShapeDtypeStruct((64, 8), jnp.float32),
    )(h2)


def kernel(pha, amp):
    pha = pha.astype(jnp.float32)
    amp = amp.astype(jnp.float32)
    pha_t = pha.transpose(3, 0, 1, 2, 4).reshape(_N_TASKS, _T)
    amp_t = amp.transpose(3, 0, 1, 2, 4).reshape(_N_SBC, 8, _T)
    cutoffs = jnp.linspace(-np.pi, np.pi, _N_BINS + 1).astype(jnp.float32)
    cut_pad = jnp.concatenate([cutoffs, jnp.zeros((5,), jnp.float32)])
    hist = _sc_hist(pha_t, amp_t, cut_pad)
    mi = _tc_finish(hist)
    return mi.reshape(2, 4, 8, 8)


# speedup vs baseline: 1.3041x; 1.2144x over previous
import functools
import math

import jax
import jax.numpy as jnp
import numpy as np
from jax import lax
from jax.experimental import pallas as pl
from jax.experimental.pallas import tpu as pltpu
from jax.experimental.pallas import tpu_sc as plsc

_N_BINS = 18
_EPS = 1e-9
_T = 1024
_LANES = 16
_CHUNKS = _T // _LANES
_N_ROWS = 9
_ROW_WORDS = _N_ROWS * _N_BINS
_LANE_STRIDE = 256
_TASK_WORDS = _LANES * _LANE_STRIDE
_N_TASKS = 128
_N_SBC = 16
_TASKS_PER_W = _N_TASKS // 32
_UNROLL = 4


def _sc_hist_kernel(pha_hbm, amp_hbm, cut_hbm, out_hbm,
                    pha_v, amp_v, hist_v, cut_v, sem):
    nc = 2
    wid = lax.axis_index("s") * nc + lax.axis_index("c")
    base_task = wid * _TASKS_PER_W
    sbc = base_task // 8

    pltpu.sync_copy(cut_hbm, cut_v)
    pltpu.sync_copy(amp_hbm.at[sbc], amp_v)
    pltpu.sync_copy(pha_hbm.at[pl.ds(base_task, _TASKS_PER_W)], pha_v)

    lane_off = lax.iota(jnp.int32, _LANES) * _LANE_STRIDE
    scale = jnp.float32(_N_BINS / (2.0 * math.pi))
    pi32 = jnp.float32(math.pi)
    ones = jnp.full((_LANES,), 1.0, dtype=jnp.float32)
    zeros = jnp.zeros((_LANES,), dtype=jnp.float32)

    copies = []
    for tt in range(_TASKS_PER_W):
        t_off = tt * _TASK_WORDS

        @plsc.parallel_loop(0, _LANES, unroll=2)
        def _zero_body(i, t_off=t_off):
            off = t_off + i * _LANE_STRIDE
            for k in range(11):
                hist_v[pl.ds(off + k * _LANES, _LANES)] = zeros

        base_v = lane_off + t_off

        def _one_chunk(i, tt=tt, base_v=base_v):
            ph = pha_v[tt, pl.ds(i * _LANES, _LANES)]
            raw = (ph + pi32) * scale
            raw = jnp.minimum(jnp.maximum(raw, 0.0), 17.0)
            idx0 = raw.astype(jnp.int32)
            c_lo = plsc.load_gather(cut_v, [idx0])
            c_hi = plsc.load_gather(cut_v, [idx0 + 1])
            dec = jnp.logical_and(ph <= c_lo, idx0 > 0).astype(jnp.int32)
            inc = jnp.logical_and(ph > c_hi, idx0 < 17).astype(jnp.int32)
            base = base_v + (idx0 - dec + inc)
            for fa in range(8):
                av = amp_v[fa, pl.ds(i * _LANES, _LANES)]
                plsc.addupdate_scatter(hist_v, [base + fa * _N_BINS], av)
            plsc.addupdate_scatter(hist_v, [base + 8 * _N_BINS], ones)

        @plsc.parallel_loop(0, _CHUNKS, unroll=_UNROLL)
        def _chunk_body(j):
            _one_chunk(j)

        copies.append(pltpu.async_copy(
            hist_v.at[pl.ds(t_off, _TASK_WORDS)],
            out_hbm.at[base_task + tt], sem))
    for cp in copies:
        cp.wait()


def _sc_hist(pha_t, amp_t, cut_pad):
    mesh = plsc.VectorSubcoreMesh(core_axis_name="c", subcore_axis_name="s")
    f = functools.partial(
        pl.kernel,
        mesh=mesh,
        out_type=jax.ShapeDtypeStruct((_N_TASKS, _TASK_WORDS), jnp.float32),
        scratch_types=[
            pltpu.VMEM((_TASKS_PER_W, _T), jnp.float32),
            pltpu.VMEM((8, _T), jnp.float32),
            pltpu.VMEM((_TASKS_PER_W * _TASK_WORDS,), jnp.float32),
            pltpu.VMEM((24,), jnp.float32),
            pltpu.SemaphoreType.DMA,
        ],
        compiler_params=pltpu.CompilerParams(needs_layout_passes=False),
    )(_sc_hist_kernel)
    return f(pha_t, amp_t, cut_pad)


def _tc_finish_kernel(h_ref, o_ref):
    h = h_ref[...]
    acc = h[:, 0:_ROW_WORDS]
    for l in range(1, _LANES):
        off = l * _LANE_STRIDE
        acc = acc + h[:, off:off + _ROW_WORDS]
    counts = acc[:, 8 * _N_BINS:9 * _N_BINS]
    log_n = jnp.float32(np.log(float(_N_BINS)))
    cols = []
    for fa in range(8):
        s_fa = acc[:, fa * _N_BINS:(fa + 1) * _N_BINS]
        mean = s_fa / (counts + _EPS)
        tot = jnp.sum(mean, axis=-1, keepdims=True)
        probs = mean / (tot + _EPS)
        kl = jnp.sum(probs * jnp.log(probs + _EPS), axis=-1, keepdims=True)
        cols.append((log_n + kl) / log_n)
    mi = jnp.concatenate(cols, axis=-1)
    mi = 0.5 * (mi[0:64, :] + mi[64:128, :])
    o_ref[...] = jnp.nan_to_num(mi, nan=0.0)


def _tc_finish(h2):
    return pl.pallas_call(
        _tc_finish_kernel,
        out_shape=jax.ShapeDtypeStruct((64, 8), jnp.float32),
    )(h2)


def kernel(pha, amp):
    pha = pha.astype(jnp.float32)
    amp = amp.astype(jnp.float32)
    pha_t = pha.transpose(3, 0, 1, 2, 4).reshape(_N_TASKS, _T)
    amp_t = amp.transpose(3, 0, 1, 2, 4).reshape(_N_SBC, 8, _T)
    cutoffs = jnp.linspace(-np.pi, np.pi, _N_BINS + 1).astype(jnp.float32)
    cut_pad = jnp.concatenate([cutoffs, jnp.zeros((5,), jnp.float32)])
    hist = _sc_hist(pha_t, amp_t, cut_pad)
    mi = _tc_finish(hist)
    return mi.reshape(2, 4, 8, 8)


# speedup vs baseline: 1.3397x; 1.0273x over previous
import functools
import math

import jax
import jax.numpy as jnp
import numpy as np
from jax import lax
from jax.experimental import pallas as pl
from jax.experimental.pallas import tpu as pltpu
from jax.experimental.pallas import tpu_sc as plsc

_N_BINS = 18
_EPS = 1e-9
_T = 1024
_LANES = 16
_CHUNKS = _T // _LANES
_N_ROWS = 9
_ROW_WORDS = _N_ROWS * _N_BINS
_LANE_STRIDE = 256
_TASK_WORDS = _LANES * _LANE_STRIDE
_N_TASKS = 128
_N_SBC = 16
_TASKS_PER_W = _N_TASKS // 32
_UNROLL = 4


def _sc_hist_kernel(x_hbm, lut_hbm, out_hbm,
                    pha_v, amp_v, hist_v, lo_v, hi_v, sem):
    nc = 2
    wid = lax.axis_index("s") * nc + lax.axis_index("c")
    base_task = wid * _TASKS_PER_W
    sbc = base_task // 8
    f_off = base_task - sbc * 8

    pltpu.sync_copy(lut_hbm.at[pl.ds(0, 24)], lo_v)
    pltpu.sync_copy(lut_hbm.at[pl.ds(24, 24)], hi_v)
    pltpu.sync_copy(x_hbm.at[pl.ds(sbc * 16 + 8, 8)], amp_v)
    pltpu.sync_copy(x_hbm.at[pl.ds(sbc * 16 + f_off, _TASKS_PER_W)], pha_v)

    lane_off = lax.iota(jnp.int32, _LANES) * _LANE_STRIDE
    scale = jnp.float32(_N_BINS / (2.0 * math.pi))
    pi32 = jnp.float32(math.pi)
    ones = jnp.full((_LANES,), 1.0, dtype=jnp.float32)
    zeros = jnp.zeros((_LANES,), dtype=jnp.float32)

    copies = []
    for tt in range(_TASKS_PER_W):
        t_off = tt * _TASK_WORDS

        @plsc.parallel_loop(0, _LANES, unroll=2)
        def _zero_body(i, t_off=t_off):
            off = t_off + i * _LANE_STRIDE
            for k in range(11):
                hist_v[pl.ds(off + k * _LANES, _LANES)] = zeros

        base_v = lane_off + t_off

        def _one_chunk(i, tt=tt, base_v=base_v):
            ph = pha_v[tt, pl.ds(i * _LANES, _LANES)]
            raw = (ph + pi32) * scale
            raw = jnp.minimum(jnp.maximum(raw, 0.0), 17.0)
            idx0 = raw.astype(jnp.int32)
            c_lo = plsc.load_gather(lo_v, [idx0])
            c_hi = plsc.load_gather(hi_v, [idx0])
            dec = (ph <= c_lo).astype(jnp.int32)
            inc = (ph > c_hi).astype(jnp.int32)
            base = base_v + (idx0 - dec + inc)
            for fa in range(8):
                av = amp_v[fa, pl.ds(i * _LANES, _LANES)]
                plsc.addupdate_scatter(hist_v, [base + fa * _N_BINS], av)
            plsc.addupdate_scatter(hist_v, [base + 8 * _N_BINS], ones)

        @plsc.parallel_loop(0, _CHUNKS, unroll=_UNROLL)
        def _chunk_body(j):
            _one_chunk(j)

        copies.append(pltpu.async_copy(
            hist_v.at[pl.ds(t_off, _TASK_WORDS)],
            out_hbm.at[base_task + tt], sem))
    for cp in copies:
        cp.wait()


def _sc_hist(x_t, lut):
    mesh = plsc.VectorSubcoreMesh(core_axis_name="c", subcore_axis_name="s")
    f = functools.partial(
        pl.kernel,
        mesh=mesh,
        out_type=jax.ShapeDtypeStruct((_N_TASKS, _TASK_WORDS), jnp.float32),
        scratch_types=[
            pltpu.VMEM((_TASKS_PER_W, _T), jnp.float32),
            pltpu.VMEM((8, _T), jnp.float32),
            pltpu.VMEM((_TASKS_PER_W * _TASK_WORDS,), jnp.float32),
            pltpu.VMEM((24,), jnp.float32),
            pltpu.VMEM((24,), jnp.float32),
            pltpu.SemaphoreType.DMA,
        ],
        compiler_params=pltpu.CompilerParams(needs_layout_passes=False),
    )(_sc_hist_kernel)
    return f(x_t, lut)


def _tc_finish_kernel(h_ref, o_ref):
    h = h_ref[...]
    acc = h[:, 0:_ROW_WORDS]
    for l in range(1, _LANES):
        off = l * _LANE_STRIDE
        acc = acc + h[:, off:off + _ROW_WORDS]
    counts = acc[:, 8 * _N_BINS:9 * _N_BINS]
    log_n = jnp.float32(np.log(float(_N_BINS)))
    cols = []
    for fa in range(8):
        s_fa = acc[:, fa * _N_BINS:(fa + 1) * _N_BINS]
        mean = s_fa / (counts + _EPS)
        tot = jnp.sum(mean, axis=-1, keepdims=True)
        probs = mean / (tot + _EPS)
        kl = jnp.sum(probs * jnp.log(probs + _EPS), axis=-1, keepdims=True)
        cols.append((log_n + kl) / log_n)
    mi = jnp.concatenate(cols, axis=-1)
    mi = 0.5 * (mi[0:64, :] + mi[64:128, :])
    o_ref[...] = jnp.nan_to_num(mi, nan=0.0)


def _tc_finish(h2):
    return pl.pallas_call(
        _tc_finish_kernel,
        out_shape=jax.ShapeDtypeStruct((64, 8), jnp.float32),
    )(h2)


def kernel(pha, amp):
    pha = pha.astype(jnp.float32)
    amp = amp.astype(jnp.float32)
    x = jnp.concatenate([pha, amp], axis=2)
    x_t = x.transpose(3, 0, 1, 2, 4).reshape(_N_SBC * 16, _T)
    cutoffs = np.linspace(-np.pi, np.pi, _N_BINS + 1).astype(np.float32)
    lo_t = np.full((24,), -np.inf, np.float32)
    lo_t[:18] = cutoffs[:18]
    lo_t[0] = -np.inf
    hi_t = np.full((24,), np.inf, np.float32)
    hi_t[:18] = cutoffs[1:19]
    hi_t[17] = np.inf
    lut = jnp.asarray(np.concatenate([lo_t, hi_t]))
    hist = _sc_hist(x_t, lut)
    mi = _tc_finish(hist)
    return mi.reshape(2, 4, 8, 8)
